# baseline (speedup 1.0000x reference)
import functools

import jax
import jax.numpy as jnp
from jax import lax
from jax.experimental import pallas as pl
from jax.experimental.pallas import tpu as pltpu
from jax.experimental.pallas import tpu_sc as plsc

K = 32
M = 128
T = 32
TJ = 1024
SEGJ = 32
NT_PAD = 32
NEG_INF = float("-inf")
IMAX = (1 << 31) - 1


def _tc1_body(x_ref, wp_ref, logits_ref, seg_ref, sm_ref, smn_ref,
              *, n_valid, nt_total):
    nt = pl.program_id(1)
    out = lax.dot_general(wp_ref[...], x_ref[0], (((1,), (1,)), ((), ())),
                          preferred_element_type=jnp.float32)
    h = (lax.broadcasted_iota(jnp.int32, (2 * M, TJ), 0) >= M).astype(jnp.int32)
    n_glob = 2 * (nt * TJ + lax.broadcasted_iota(jnp.int32, (2 * M, TJ), 1)) + h
    out = jnp.where(n_glob < n_valid, out, NEG_INF)
    logits_ref[0, 0] = out

    @pl.when(nt == 0)
    def _init():
        sm_ref[...] = jnp.full((2 * NT_PAD, M, TJ // SEGJ), NEG_INF, jnp.float32)

    spt = TJ // SEGJ
    o3 = out.reshape(2 * M, spt, SEGJ)
    n3 = n_glob.reshape(2 * M, spt, SEGJ)
    smax = jnp.max(o3, axis=2)
    snarg = jnp.min(jnp.where(o3 == smax[:, :, None], n3, IMAX), axis=2)
    sm_ref[pl.ds(2 * nt, 2)] = smax.reshape(2, M, spt)
    smn_ref[pl.ds(2 * nt, 2)] = snarg.reshape(2, M, spt)

    @pl.when(nt == nt_total - 1)
    def _select_segments():
        v = sm_ref[...]
        nkey = smn_ref[...]
        rank = lax.broadcasted_iota(jnp.int32, (M, K), 1)

        def ext(k, carry):
            v, out_s = carry
            m1 = jnp.max(jnp.max(v, axis=2, keepdims=True), axis=0,
                         keepdims=True)
            sel = jnp.where(v == m1, nkey, IMAX)
            i1 = jnp.min(jnp.min(sel, axis=2, keepdims=True), axis=0,
                         keepdims=True)
            v = jnp.where((v == m1) & (nkey == i1), NEG_INF, v)
            out_s = jnp.where(rank == k, i1[0], out_s)
            return v, out_s

        _, out_s = lax.fori_loop(0, K, ext, (v, jnp.zeros((M, K), jnp.int32)))
        seg_ref[0] = out_s


def _tc1(xr, wp, n_valid):
    b, nj, t4 = xr.shape
    nt_total = (nj + TJ - 1) // TJ
    return pl.pallas_call(
        functools.partial(_tc1_body, n_valid=n_valid, nt_total=nt_total),
        grid=(b, nt_total),
        in_specs=[
            pl.BlockSpec((1, TJ, t4), lambda bi, ni: (bi, ni, 0)),
            pl.BlockSpec((2 * M, t4), lambda bi, ni: (0, 0)),
        ],
        out_specs=[
            pl.BlockSpec((1, 1, 2 * M, TJ), lambda bi, ni: (bi, ni, 0, 0)),
            pl.BlockSpec((1, M, K), lambda bi, ni: (bi, 0, 0)),
        ],
        out_shape=[
            jax.ShapeDtypeStruct((b, nt_total, 2 * M, TJ), jnp.float32),
            jax.ShapeDtypeStruct((b, M, K), jnp.int32),
        ],
        scratch_shapes=[
            pltpu.VMEM((2 * NT_PAD, M, TJ // SEGJ), jnp.float32),
            pltpu.VMEM((2 * NT_PAD, M, TJ // SEGJ), jnp.int32),
        ],
    )(xr, wp)


def _tc2_body(cv_ref, cn_ref, out_ref):
    v = cv_ref[0]
    n = cn_ref[0]
    rank = lax.broadcasted_iota(jnp.int32, (M, K), 1)

    def ext(k, carry):
        v, out_i = carry
        m1 = jnp.max(v, axis=1, keepdims=True)
        i1 = jnp.min(jnp.where(v == m1, n, IMAX), axis=1, keepdims=True)
        v = jnp.where(n == i1, NEG_INF, v)
        out_i = jnp.where(rank == k, i1, out_i)
        return v, out_i

    _, out_i = lax.fori_loop(0, K, ext,
                             (v, jnp.zeros((M, K), jnp.int32)))
    out_ref[0] = out_i


def _tc2(cand_v, cand_n):
    b = cand_v.shape[0]
    nc = cand_v.shape[2]
    return pl.pallas_call(
        _tc2_body,
        grid=(b,),
        in_specs=[
            pl.BlockSpec((1, M, nc), lambda bi: (bi, 0, 0)),
            pl.BlockSpec((1, M, nc), lambda bi: (bi, 0, 0)),
        ],
        out_specs=pl.BlockSpec((1, M, K), lambda bi: (bi, 0, 0)),
        out_shape=jax.ShapeDtypeStruct((b, M, K), jnp.int32),
    )(cand_v, cand_n)


def _sc_gather1(table, flat_idx):
    num_rows, d = table.shape
    num_idx = flat_idx.shape[0]
    info = plsc.get_sparse_core_info()
    nw = info.num_cores * info.num_subcores
    per_w = num_idx // nw
    mesh = plsc.VectorSubcoreMesh(core_axis_name="c", subcore_axis_name="s")

    @functools.partial(
        pl.kernel, mesh=mesh,
        compiler_params=pltpu.CompilerParams(use_tc_tiling_on_sc=False),
        out_type=jax.ShapeDtypeStruct((num_idx, d), jnp.float32),
        scratch_types=[
            pltpu.VMEM((per_w,), jnp.int32),
            pltpu.VMEM((per_w, d), jnp.float32),
            pltpu.SemaphoreType.DMA,
        ],
    )
    def gk(t_hbm, idx_hbm, o_hbm, idx_v, r_v, sem):
        wid = lax.axis_index("s") * info.num_cores + lax.axis_index("c")
        base = wid * per_w
        pltpu.sync_copy(idx_hbm.at[pl.ds(base, per_w)], idx_v)
        pltpu.async_copy(t_hbm.at[idx_v], r_v, sem).wait()
        pltpu.sync_copy(r_v, o_hbm.at[pl.ds(base, per_w)])

    return gk(table, flat_idx)


def _sc_gather2(table1, table2, flat_idx):
    num_rows, d = table1.shape
    num_idx = flat_idx.shape[0]
    info = plsc.get_sparse_core_info()
    nw = info.num_cores * info.num_subcores
    per_w = num_idx // nw
    mesh = plsc.VectorSubcoreMesh(core_axis_name="c", subcore_axis_name="s")

    @functools.partial(
        pl.kernel, mesh=mesh,
        compiler_params=pltpu.CompilerParams(use_tc_tiling_on_sc=False),
        out_type=(jax.ShapeDtypeStruct((num_idx, d), jnp.float32),
                  jax.ShapeDtypeStruct((num_idx, d), jnp.float32)),
        scratch_types=[
            pltpu.VMEM((per_w,), jnp.int32),
            pltpu.VMEM((per_w, d), jnp.float32),
            pltpu.VMEM((per_w, d), jnp.float32),
            pltpu.SemaphoreType.DMA,
        ],
    )
    def gk(t1_hbm, t2_hbm, idx_hbm, o1_hbm, o2_hbm, idx_v, r1_v, r2_v, sem):
        wid = lax.axis_index("s") * info.num_cores + lax.axis_index("c")
        base = wid * per_w
        pltpu.sync_copy(idx_hbm.at[pl.ds(base, per_w)], idx_v)
        pltpu.async_copy(t1_hbm.at[idx_v], r1_v, sem).wait()
        pltpu.async_copy(t2_hbm.at[idx_v], r2_v, sem).wait()
        pltpu.sync_copy(r1_v, o1_hbm.at[pl.ds(base, per_w)])
        pltpu.sync_copy(r2_v, o2_hbm.at[pl.ds(base, per_w)])

    return gk(table1, table2, flat_idx)


def kernel(nodevec1, nodevec2, node_embeddings):
    b, n, t = nodevec1.shape
    nj = n // 2
    xr = jnp.concatenate([nodevec1.reshape(b, nj, 2 * t),
                          nodevec2.reshape(b, nj, 2 * t)], axis=2)
    emb = node_embeddings
    wp = jnp.zeros((2 * M, 4 * t), jnp.float32)
    wp = wp.at[:M, 0:t].set(emb[:, :t]).at[:M, 2 * t:3 * t].set(emb[:, t:])
    wp = wp.at[M:, t:2 * t].set(emb[:, :t]).at[M:, 3 * t:4 * t].set(emb[:, t:])

    logits, nmax = _tc1(xr, wp, n)
    spt = TJ // SEGJ
    nt_total = (nj + TJ - 1) // TJ
    b_idx = jnp.arange(b, dtype=jnp.int32)[:, None, None]
    m_idx = jnp.arange(M, dtype=jnp.int32)[None, :, None]
    h = nmax & 1
    jg = nmax >> 1
    nt_of = jg // TJ
    s2 = (jg // SEGJ) % spt
    flat_seg = (((b_idx * nt_total + nt_of) * (2 * M) + h * M + m_idx) * spt
                + s2).reshape(-1)
    cand = _sc_gather1(
        logits.reshape(b * nt_total * 2 * M * spt, SEGJ), flat_seg)
    cand_v = cand.reshape(b, M, K * SEGJ)
    base = 2 * ((jg // SEGJ) * SEGJ) + h
    cand_n = (base[..., None]
              + 2 * jnp.arange(SEGJ, dtype=jnp.int32)).reshape(b, M, K * SEGJ)

    indices = _tc2(cand_v, cand_n)

    flat_idx = (indices + jnp.arange(b, dtype=jnp.int32)[:, None, None] * n
                ).reshape(-1)
    f1, f2 = _sc_gather2(nodevec1.reshape(b * n, t),
                         nodevec2.reshape(b * n, t), flat_idx)
    sel1 = f1.reshape(b, M, K, t)
    sel2 = f2.reshape(b, M, K, t)
    batch_indices = jnp.broadcast_to(
        jnp.arange(b, dtype=jnp.int32)[:, None, None], (b, M, K))
    return sel1, sel2, batch_indices, indices


# speedup vs baseline: 1.8260x; 1.8260x over previous
import functools

import jax
import jax.numpy as jnp
from jax import lax
from jax.experimental import pallas as pl
from jax.experimental.pallas import tpu as pltpu
from jax.experimental.pallas import tpu_sc as plsc

K = 32
M = 128
T = 32
TILE = 2048
SEG = 32
NT_PAD = 32
NEG_INF = float("-inf")
IMAX = (1 << 31) - 1


def _tc1_body(nv1_ref, nv2_ref, emb_ref, logits_ref, seg_ref, sm_ref, *,
              n_valid, nt_total):
    nt = pl.program_id(1)
    nv3 = jnp.concatenate([nv1_ref[0], nv2_ref[0]], axis=1)
    half = TILE // 2
    emb = emb_ref[...]
    la = lax.dot_general(emb, nv3[:half], (((1,), (1,)), ((), ())),
                         preferred_element_type=jnp.float32)
    lb = lax.dot_general(emb, nv3[half:], (((1,), (1,)), ((), ())),
                         preferred_element_type=jnp.float32)
    ng = lax.broadcasted_iota(jnp.int32, (M, half), 1)
    la = jnp.where(nt * TILE + ng < n_valid, la, NEG_INF)
    lb = jnp.where(nt * TILE + half + ng < n_valid, lb, NEG_INF)
    logits_ref[0, 0, :, :half] = la
    logits_ref[0, 0, :, half:] = lb

    @pl.when(nt == 0)
    def _init():
        sm_ref[...] = jnp.full((NT_PAD, M, TILE // SEG), NEG_INF, jnp.float32)

    sm_ref[nt] = jnp.concatenate(
        [jnp.max(la.reshape(M, half // SEG, SEG), axis=2),
         jnp.max(lb.reshape(M, half // SEG, SEG), axis=2)], axis=1)

    @pl.when(nt == nt_total - 1)
    def _select_segments():
        spt = TILE // SEG
        v = sm_ref[...]
        seg_iota = (lax.broadcasted_iota(jnp.int32, (NT_PAD, M, spt), 0) * spt
                    + lax.broadcasted_iota(jnp.int32, (NT_PAD, M, spt), 2))
        rank = lax.broadcasted_iota(jnp.int32, (M, K), 1)

        def ext(k, carry):
            v, out_s = carry
            m1 = jnp.max(jnp.max(v, axis=2, keepdims=True), axis=0,
                         keepdims=True)
            sid = jnp.where(v == m1, seg_iota, IMAX)
            i1 = jnp.min(jnp.min(sid, axis=2, keepdims=True), axis=0,
                         keepdims=True)
            v = jnp.where(seg_iota == i1, NEG_INF, v)
            out_s = jnp.where(rank == k, i1[0], out_s)
            return v, out_s

        _, out_s = lax.fori_loop(0, K, ext, (v, jnp.zeros((M, K), jnp.int32)))
        seg_ref[0] = out_s


def _tc1(nv1, nv2, emb, n_valid):
    b, n, t = nv1.shape
    nt_total = (n + TILE - 1) // TILE
    n_pad = nt_total * TILE
    return pl.pallas_call(
        functools.partial(_tc1_body, n_valid=n_valid, nt_total=nt_total),
        grid=(b, nt_total),
        in_specs=[
            pl.BlockSpec((1, TILE, t), lambda bi, ni: (bi, ni, 0)),
            pl.BlockSpec((1, TILE, t), lambda bi, ni: (bi, ni, 0)),
            pl.BlockSpec((M, 2 * t), lambda bi, ni: (0, 0)),
        ],
        out_specs=[
            pl.BlockSpec((1, 1, M, TILE), lambda bi, ni: (bi, ni, 0, 0)),
            pl.BlockSpec((1, M, K), lambda bi, ni: (bi, 0, 0)),
        ],
        out_shape=[
            jax.ShapeDtypeStruct((b, nt_total, M, TILE), jnp.float32),
            jax.ShapeDtypeStruct((b, M, K), jnp.int32),
        ],
        scratch_shapes=[
            pltpu.VMEM((NT_PAD, M, TILE // SEG), jnp.float32),
        ],
    )(nv1, nv2, emb)


def _tc2_body(cv_ref, cn_ref, out_ref):
    v = cv_ref[0]
    n = cn_ref[0]
    rank = lax.broadcasted_iota(jnp.int32, (M, K), 1)

    def ext(k, carry):
        v, out_i = carry
        m1 = jnp.max(v, axis=1, keepdims=True)
        i1 = jnp.min(jnp.where(v == m1, n, IMAX), axis=1, keepdims=True)
        v = jnp.where(n == i1, NEG_INF, v)
        out_i = jnp.where(rank == k, i1, out_i)
        return v, out_i

    _, out_i = lax.fori_loop(0, K, ext,
                             (v, jnp.zeros((M, K), jnp.int32)))
    out_ref[0] = out_i


def _tc2(cand_v, cand_n):
    b = cand_v.shape[0]
    nc = cand_v.shape[2]
    return pl.pallas_call(
        _tc2_body,
        grid=(b,),
        in_specs=[
            pl.BlockSpec((1, M, nc), lambda bi: (bi, 0, 0)),
            pl.BlockSpec((1, M, nc), lambda bi: (bi, 0, 0)),
        ],
        out_specs=pl.BlockSpec((1, M, K), lambda bi: (bi, 0, 0)),
        out_shape=jax.ShapeDtypeStruct((b, M, K), jnp.int32),
    )(cand_v, cand_n)


def _sc_gather1(table, flat_idx):
    num_rows, d = table.shape
    num_idx = flat_idx.shape[0]
    info = plsc.get_sparse_core_info()
    nw = info.num_cores * info.num_subcores
    per_w = num_idx // nw
    mesh = plsc.VectorSubcoreMesh(core_axis_name="c", subcore_axis_name="s")

    @functools.partial(
        pl.kernel, mesh=mesh,
        compiler_params=pltpu.CompilerParams(use_tc_tiling_on_sc=False),
        out_type=jax.ShapeDtypeStruct((num_idx, d), jnp.float32),
        scratch_types=[
            pltpu.VMEM((per_w,), jnp.int32),
            pltpu.VMEM((per_w, d), jnp.float32),
            pltpu.SemaphoreType.DMA,
        ],
    )
    def gk(t_hbm, idx_hbm, o_hbm, idx_v, r_v, sem):
        wid = lax.axis_index("s") * info.num_cores + lax.axis_index("c")
        base = wid * per_w
        pltpu.sync_copy(idx_hbm.at[pl.ds(base, per_w)], idx_v)
        pltpu.async_copy(t_hbm.at[idx_v], r_v, sem).wait()
        pltpu.sync_copy(r_v, o_hbm.at[pl.ds(base, per_w)])

    return gk(table, flat_idx)


def _sc_gather2(table1, table2, flat_idx):
    num_rows, d = table1.shape
    num_idx = flat_idx.shape[0]
    info = plsc.get_sparse_core_info()
    nw = info.num_cores * info.num_subcores
    per_w = num_idx // nw
    mesh = plsc.VectorSubcoreMesh(core_axis_name="c", subcore_axis_name="s")

    @functools.partial(
        pl.kernel, mesh=mesh,
        compiler_params=pltpu.CompilerParams(use_tc_tiling_on_sc=False),
        out_type=(jax.ShapeDtypeStruct((num_idx, d), jnp.float32),
                  jax.ShapeDtypeStruct((num_idx, d), jnp.float32)),
        scratch_types=[
            pltpu.VMEM((per_w,), jnp.int32),
            pltpu.VMEM((per_w, d), jnp.float32),
            pltpu.VMEM((per_w, d), jnp.float32),
            pltpu.SemaphoreType.DMA,
        ],
    )
    def gk(t1_hbm, t2_hbm, idx_hbm, o1_hbm, o2_hbm, idx_v, r1_v, r2_v, sem):
        wid = lax.axis_index("s") * info.num_cores + lax.axis_index("c")
        base = wid * per_w
        pltpu.sync_copy(idx_hbm.at[pl.ds(base, per_w)], idx_v)
        pltpu.async_copy(t1_hbm.at[idx_v], r1_v, sem).wait()
        pltpu.async_copy(t2_hbm.at[idx_v], r2_v, sem).wait()
        pltpu.sync_copy(r1_v, o1_hbm.at[pl.ds(base, per_w)])
        pltpu.sync_copy(r2_v, o2_hbm.at[pl.ds(base, per_w)])

    return gk(table1, table2, flat_idx)


def kernel(nodevec1, nodevec2, node_embeddings):
    b, n, t = nodevec1.shape
    n_pad = ((n + TILE - 1) // TILE) * TILE
    segs_per_row = n_pad // SEG

    logits, seg = _tc1(nodevec1, nodevec2, node_embeddings, n)

    spt = TILE // SEG
    nt_total = n_pad // TILE
    b_idx = jnp.arange(b, dtype=jnp.int32)[:, None, None]
    m_idx = jnp.arange(M, dtype=jnp.int32)[None, :, None]
    flat_seg = (((b_idx * nt_total + seg // spt) * M + m_idx) * spt
                + seg % spt).reshape(-1)
    cand = _sc_gather1(logits.reshape(b * M * segs_per_row, SEG), flat_seg)
    cand_v = cand.reshape(b, M, K * SEG)
    cand_n = (seg[..., None] * SEG
              + jnp.arange(SEG, dtype=jnp.int32)).reshape(b, M, K * SEG)

    indices = _tc2(cand_v, cand_n)

    flat_idx = (indices + jnp.arange(b, dtype=jnp.int32)[:, None, None] * n
                ).reshape(-1)
    f1, f2 = _sc_gather2(nodevec1.reshape(b * n, t),
                         nodevec2.reshape(b * n, t), flat_idx)
    sel1 = f1.reshape(b, M, K, t)
    sel2 = f2.reshape(b, M, K, t)
    batch_indices = jnp.broadcast_to(
        jnp.arange(b, dtype=jnp.int32)[:, None, None], (b, M, K))
    return sel1, sel2, batch_indices, indices


# speedup vs baseline: 1.8354x; 1.0052x over previous
import functools

import jax
import jax.numpy as jnp
from jax import lax
from jax.experimental import pallas as pl
from jax.experimental.pallas import tpu as pltpu
from jax.experimental.pallas import tpu_sc as plsc

K = 32
M = 128
T = 32
TILE = 2048
SEG = 32
NT_PAD = 32
NEG_INF = float("-inf")
IMAX = (1 << 31) - 1


def _tc1_body(nv1_ref, nv2_ref, emb_ref, logits_ref, seg_ref, sm_ref, *,
              n_valid, nt_total):
    nt = pl.program_id(1)
    nv3 = jnp.concatenate([nv1_ref[0], nv2_ref[0]], axis=1)
    logits = lax.dot_general(emb_ref[...], nv3, (((1,), (1,)), ((), ())),
                             preferred_element_type=jnp.float32)
    n_glob = nt * TILE + lax.broadcasted_iota(jnp.int32, (M, TILE), 1)
    logits = jnp.where(n_glob < n_valid, logits, NEG_INF)
    logits_ref[0, 0] = logits

    @pl.when(nt == 0)
    def _init():
        sm_ref[...] = jnp.full((NT_PAD, M, TILE // SEG), NEG_INF, jnp.float32)

    sm_ref[nt] = jnp.max(logits.reshape(M, TILE // SEG, SEG), axis=2)

    @pl.when(nt == nt_total - 1)
    def _select_segments():
        spt = TILE // SEG
        v = sm_ref[...]
        seg_iota = (lax.broadcasted_iota(jnp.int32, (NT_PAD, M, spt), 0) * spt
                    + lax.broadcasted_iota(jnp.int32, (NT_PAD, M, spt), 2))
        rank = lax.broadcasted_iota(jnp.int32, (M, K), 1)

        def ext(k, carry):
            v, out_s = carry
            m1 = jnp.max(jnp.max(v, axis=2, keepdims=True), axis=0,
                         keepdims=True)
            sid = jnp.where(v == m1, seg_iota, IMAX)
            i1 = jnp.min(jnp.min(sid, axis=2, keepdims=True), axis=0,
                         keepdims=True)
            v = jnp.where(seg_iota == i1, NEG_INF, v)
            out_s = jnp.where(rank == k, i1[0], out_s)
            return v, out_s

        _, out_s = lax.fori_loop(0, K, ext, (v, jnp.zeros((M, K), jnp.int32)))
        seg_ref[0] = out_s


def _tc1(nv1, nv2, emb, n_valid):
    b, n, t = nv1.shape
    nt_total = (n + TILE - 1) // TILE
    n_pad = nt_total * TILE
    return pl.pallas_call(
        functools.partial(_tc1_body, n_valid=n_valid, nt_total=nt_total),
        grid=(b, nt_total),
        in_specs=[
            pl.BlockSpec((1, TILE, t), lambda bi, ni: (bi, ni, 0)),
            pl.BlockSpec((1, TILE, t), lambda bi, ni: (bi, ni, 0)),
            pl.BlockSpec((M, 2 * t), lambda bi, ni: (0, 0)),
        ],
        out_specs=[
            pl.BlockSpec((1, 1, M, TILE), lambda bi, ni: (bi, ni, 0, 0)),
            pl.BlockSpec((1, M, K), lambda bi, ni: (bi, 0, 0)),
        ],
        out_shape=[
            jax.ShapeDtypeStruct((b, nt_total, M, TILE), jnp.float32),
            jax.ShapeDtypeStruct((b, M, K), jnp.int32),
        ],
        scratch_shapes=[
            pltpu.VMEM((NT_PAD, M, TILE // SEG), jnp.float32),
        ],
    )(nv1, nv2, emb)


def _tc2_body(cv_ref, cn_ref, out_ref):
    v = cv_ref[0]
    n = cn_ref[0]
    rank = lax.broadcasted_iota(jnp.int32, (M, K), 1)

    def ext(k, carry):
        v, out_i = carry
        m1 = jnp.max(v, axis=1, keepdims=True)
        i1 = jnp.min(jnp.where(v == m1, n, IMAX), axis=1, keepdims=True)
        v = jnp.where(n == i1, NEG_INF, v)
        out_i = jnp.where(rank == k, i1, out_i)
        return v, out_i

    _, out_i = lax.fori_loop(0, K, ext,
                             (v, jnp.zeros((M, K), jnp.int32)))
    out_ref[0] = out_i


def _tc2(cand_v, cand_n):
    b = cand_v.shape[0]
    nc = cand_v.shape[2]
    return pl.pallas_call(
        _tc2_body,
        grid=(b,),
        in_specs=[
            pl.BlockSpec((1, M, nc), lambda bi: (bi, 0, 0)),
            pl.BlockSpec((1, M, nc), lambda bi: (bi, 0, 0)),
        ],
        out_specs=pl.BlockSpec((1, M, K), lambda bi: (bi, 0, 0)),
        out_shape=jax.ShapeDtypeStruct((b, M, K), jnp.int32),
    )(cand_v, cand_n)


def _sc_gather1(table, flat_idx):
    num_rows, d = table.shape
    num_idx = flat_idx.shape[0]
    info = plsc.get_sparse_core_info()
    nw = info.num_cores * info.num_subcores
    per_w = num_idx // nw
    mesh = plsc.VectorSubcoreMesh(core_axis_name="c", subcore_axis_name="s")

    @functools.partial(
        pl.kernel, mesh=mesh,
        compiler_params=pltpu.CompilerParams(use_tc_tiling_on_sc=False),
        out_type=jax.ShapeDtypeStruct((num_idx, d), jnp.float32),
        scratch_types=[
            pltpu.VMEM((per_w,), jnp.int32),
            pltpu.VMEM((per_w, d), jnp.float32),
            pltpu.SemaphoreType.DMA,
        ],
    )
    def gk(t_hbm, idx_hbm, o_hbm, idx_v, r_v, sem):
        wid = lax.axis_index("s") * info.num_cores + lax.axis_index("c")
        base = wid * per_w
        pltpu.sync_copy(idx_hbm.at[pl.ds(base, per_w)], idx_v)
        pltpu.async_copy(t_hbm.at[idx_v], r_v, sem).wait()
        pltpu.sync_copy(r_v, o_hbm.at[pl.ds(base, per_w)])

    return gk(table, flat_idx)


def _sc_gather2(table1, table2, flat_idx):
    num_rows, d = table1.shape
    num_idx = flat_idx.shape[0]
    info = plsc.get_sparse_core_info()
    nw = info.num_cores * info.num_subcores
    per_w = num_idx // nw
    mesh = plsc.VectorSubcoreMesh(core_axis_name="c", subcore_axis_name="s")

    @functools.partial(
        pl.kernel, mesh=mesh,
        compiler_params=pltpu.CompilerParams(use_tc_tiling_on_sc=False),
        out_type=(jax.ShapeDtypeStruct((num_idx, d), jnp.float32),
                  jax.ShapeDtypeStruct((num_idx, d), jnp.float32)),
        scratch_types=[
            pltpu.VMEM((per_w,), jnp.int32),
            pltpu.VMEM((per_w, d), jnp.float32),
            pltpu.VMEM((per_w, d), jnp.float32),
            pltpu.SemaphoreType.DMA,
        ],
    )
    def gk(t1_hbm, t2_hbm, idx_hbm, o1_hbm, o2_hbm, idx_v, r1_v, r2_v, sem):
        wid = lax.axis_index("s") * info.num_cores + lax.axis_index("c")
        base = wid * per_w
        pltpu.sync_copy(idx_hbm.at[pl.ds(base, per_w)], idx_v)
        pltpu.async_copy(t1_hbm.at[idx_v], r1_v, sem).wait()
        pltpu.async_copy(t2_hbm.at[idx_v], r2_v, sem).wait()
        pltpu.sync_copy(r1_v, o1_hbm.at[pl.ds(base, per_w)])
        pltpu.sync_copy(r2_v, o2_hbm.at[pl.ds(base, per_w)])

    return gk(table1, table2, flat_idx)


def kernel(nodevec1, nodevec2, node_embeddings):
    b, n, t = nodevec1.shape
    n_pad = ((n + TILE - 1) // TILE) * TILE
    segs_per_row = n_pad // SEG

    logits, seg = _tc1(nodevec1, nodevec2, node_embeddings, n)

    spt = TILE // SEG
    nt_total = n_pad // TILE
    b_idx = jnp.arange(b, dtype=jnp.int32)[:, None, None]
    m_idx = jnp.arange(M, dtype=jnp.int32)[None, :, None]
    flat_seg = (((b_idx * nt_total + seg // spt) * M + m_idx) * spt
                + seg % spt).reshape(-1)
    cand = _sc_gather1(logits.reshape(b * M * segs_per_row, SEG), flat_seg)
    cand_v = cand.reshape(b, M, K * SEG)
    cand_n = (seg[..., None] * SEG
              + jnp.arange(SEG, dtype=jnp.int32)).reshape(b, M, K * SEG)

    indices = _tc2(cand_v, cand_n)

    flat_idx = (indices + jnp.arange(b, dtype=jnp.int32)[:, None, None] * n
                ).reshape(-1)
    f1, f2 = _sc_gather2(nodevec1.reshape(b * n, t),
                         nodevec2.reshape(b * n, t), flat_idx)
    sel1 = f1.reshape(b, M, K, t)
    sel2 = f2.reshape(b, M, K, t)
    batch_indices = jnp.broadcast_to(
        jnp.arange(b, dtype=jnp.int32)[:, None, None], (b, M, K))
    return sel1, sel2, batch_indices, indices
